# baseline (speedup 1.0000x reference)
import jax
import jax.numpy as jnp
from jax import lax
from jax.experimental import pallas as pl
from jax.experimental.pallas import tpu as pltpu
from jax.experimental.pallas import tpu_sc as plsc

BATCH = 16384
EMBED_DIM = 32
ROW_PACK = 4
SUPER = EMBED_DIM * ROW_PACK
NUM_CORES = 2
NUM_SUBCORES = 16
NUM_WORKERS = NUM_CORES * NUM_SUBCORES
B_PER_W = BATCH // NUM_WORKERS
LANES = 16
CHUNK = 256
N_CHUNKS = B_PER_W // CHUNK
GROUPS = CHUNK // LANES


def _body(user_ids_hbm, item_ids_hbm, user_table_hbm, item_table_hbm,
          out_hbm, uid_v, iid_v, uidx_v, iidx_v, urows_v, irows_v, out_v,
          sem):
    wid = lax.axis_index("s") * NUM_CORES + lax.axis_index("c")
    base = wid * B_PER_W

    pltpu.sync_copy(user_ids_hbm.at[pl.ds(base, B_PER_W)], uid_v)
    pltpu.sync_copy(item_ids_hbm.at[pl.ds(base, B_PER_W)], iid_v)

    lane = lax.iota(jnp.int32, LANES)

    def chunk_body(c, carry):
        c0 = c * CHUNK
        def idx_body(k, carry2):
            s = pl.ds(c0 + k * LANES, LANES)
            d = pl.ds(k * LANES, LANES)
            uidx_v[d] = uid_v[s] >> 2
            iidx_v[d] = iid_v[s] >> 2
            return carry2
        lax.fori_loop(0, CHUNK // LANES, idx_body, 0)

        cp_u = pltpu.async_copy(user_table_hbm.at[uidx_v], urows_v, sem)
        cp_i = pltpu.async_copy(item_table_hbm.at[iidx_v], irows_v, sem)
        cp_u.wait()
        cp_i.wait()

        def group(g, carry2):
            row0 = g * LANES
            rows = row0 + lane
            uoff = (uid_v[pl.ds(c0 + row0, LANES)] & 3) * EMBED_DIM
            ioff = (iid_v[pl.ds(c0 + row0, LANES)] & 3) * EMBED_DIM
            acc = jnp.zeros((LANES,), jnp.float32)
            for d in range(EMBED_DIM):
                u = plsc.load_gather(urows_v, [rows, uoff + d])
                v = plsc.load_gather(irows_v, [rows, ioff + d])
                acc = acc + u * v
            out_v[pl.ds(c0 + row0, LANES)] = acc
            return carry2

        lax.fori_loop(0, GROUPS, group, 0)
        return carry

    lax.fori_loop(0, N_CHUNKS, chunk_body, 0)

    pltpu.sync_copy(out_v, out_hbm.at[pl.ds(base, B_PER_W)])


@jax.jit
def kernel(user_ids, item_ids, user_table, item_table):
    mesh = plsc.VectorSubcoreMesh(core_axis_name="c", subcore_axis_name="s")
    f = pl.kernel(
        _body,
        mesh=mesh,
        compiler_params=pltpu.CompilerParams(needs_layout_passes=False),
        out_type=jax.ShapeDtypeStruct((BATCH,), jnp.float32),
        scratch_types=[
            pltpu.VMEM((B_PER_W,), jnp.int32),
            pltpu.VMEM((B_PER_W,), jnp.int32),
            pltpu.VMEM((CHUNK,), jnp.int32),
            pltpu.VMEM((CHUNK,), jnp.int32),
            pltpu.VMEM((CHUNK, SUPER), jnp.float32),
            pltpu.VMEM((CHUNK, SUPER), jnp.float32),
            pltpu.VMEM((B_PER_W,), jnp.float32),
            pltpu.SemaphoreType.DMA,
        ],
    )
    ut = user_table.reshape(-1, SUPER)
    it = item_table.reshape(-1, SUPER)
    return f(user_ids.astype(jnp.int32), item_ids.astype(jnp.int32), ut, it)


# speedup vs baseline: 3.9493x; 3.9493x over previous
import jax
import jax.numpy as jnp
from jax import lax
from jax.experimental import pallas as pl
from jax.experimental.pallas import tpu as pltpu
from jax.experimental.pallas import tpu_sc as plsc

BATCH = 16384
EMBED_DIM = 32
SLAB = 128
NUM_CORES = 2
NUM_SUBCORES = 16
NUM_WORKERS = NUM_CORES * NUM_SUBCORES
B_PER_W = BATCH // NUM_WORKERS
LANES = 16
GROUPS = B_PER_W // LANES
NBUF = 4


def _body(user_ids_hbm, item_ids_hbm, user_t_hbm, item_t_hbm, out_hbm,
          uid_v, iid_v, stage_v, out_v,
          ubuf0, ubuf1, ubuf2, ubuf3, vbuf0, vbuf1, vbuf2, vbuf3,
          usem0, usem1, usem2, usem3, vsem0, vsem1, vsem2, vsem3):
    wid = lax.axis_index("s") * NUM_CORES + lax.axis_index("c")
    base = wid * B_PER_W

    pltpu.sync_copy(user_ids_hbm.at[pl.ds(base, B_PER_W)], uid_v)
    pltpu.sync_copy(item_ids_hbm.at[pl.ds(base, B_PER_W)], iid_v)

    ubufs = [ubuf0, ubuf1, ubuf2, ubuf3]
    vbufs = [vbuf0, vbuf1, vbuf2, vbuf3]
    usems = [usem0, usem1, usem2, usem3]
    vsems = [vsem0, vsem1, vsem2, vsem3]

    lane = lax.iota(jnp.int32, LANES)

    def fire(slot, r_u, r_v):
        ju = pl.multiple_of((r_u >> 7) * SLAB, SLAB)
        jv = pl.multiple_of((r_v >> 7) * SLAB, SLAB)
        pltpu.make_async_copy(
            user_t_hbm.at[:, pl.ds(ju, SLAB)], ubufs[slot],
            usems[slot]).start()
        pltpu.make_async_copy(
            item_t_hbm.at[:, pl.ds(jv, SLAB)], vbufs[slot],
            vsems[slot]).start()

    def waitbuf(slot):
        pltpu.make_async_copy(
            user_t_hbm.at[:, pl.ds(0, SLAB)], ubufs[slot],
            usems[slot]).wait()
        pltpu.make_async_copy(
            item_t_hbm.at[:, pl.ds(0, SLAB)], vbufs[slot],
            vsems[slot]).wait()

    u16p = uid_v[pl.ds(0, LANES)]
    v16p = iid_v[pl.ds(0, LANES)]
    for j in range(NBUF):
        fire(j, u16p[j], v16p[j])

    def group(g, carry):
        g0 = g * LANES
        u16 = uid_v[pl.ds(g0, LANES)]
        v16 = iid_v[pl.ds(g0, LANES)]
        gn0 = jnp.minimum(g + 1, GROUPS - 1) * LANES
        u16n = uid_v[pl.ds(gn0, LANES)]
        v16n = iid_v[pl.ds(gn0, LANES)]
        for j in range(LANES):
            slot = j % NBUF
            jf = j + NBUF
            waitbuf(slot)
            cu = u16[j] & (SLAB - 1)
            cv = v16[j] & (SLAB - 1)
            cu_v = jnp.full((LANES,), cu, jnp.int32)
            cv_v = jnp.full((LANES,), cv, jnp.int32)
            ua = plsc.load_gather(ubufs[slot], [lane, cu_v])
            ub = plsc.load_gather(ubufs[slot], [lane + LANES, cu_v])
            va = plsc.load_gather(vbufs[slot], [lane, cv_v])
            vb = plsc.load_gather(vbufs[slot], [lane + LANES, cv_v])

            if jf < LANES:
                fire(slot, u16[jf], v16[jf])
            else:
                @pl.when(g + 1 < GROUPS)
                def _():
                    fire(slot, u16n[jf - LANES], v16n[jf - LANES])

            stage_v[pl.ds(j * LANES, LANES)] = ua * va + ub * vb

        acc = jnp.zeros((LANES,), jnp.float32)
        for l in range(LANES):
            acc = acc + plsc.load_gather(
                stage_v, [lane * LANES + l])
        out_v[pl.ds(g0, LANES)] = acc
        return carry

    lax.fori_loop(0, GROUPS, group, 0)

    pltpu.sync_copy(out_v, out_hbm.at[pl.ds(base, B_PER_W)])


@jax.jit
def kernel(user_ids, item_ids, user_table, item_table):
    mesh = plsc.VectorSubcoreMesh(core_axis_name="c", subcore_axis_name="s")
    f = pl.kernel(
        _body,
        mesh=mesh,
        compiler_params=pltpu.CompilerParams(needs_layout_passes=False),
        out_type=jax.ShapeDtypeStruct((BATCH,), jnp.float32),
        scratch_types=(
            [pltpu.VMEM((B_PER_W,), jnp.int32),
             pltpu.VMEM((B_PER_W,), jnp.int32),
             pltpu.VMEM((LANES * LANES,), jnp.float32),
             pltpu.VMEM((B_PER_W,), jnp.float32)]
            + [pltpu.VMEM((EMBED_DIM, SLAB), jnp.float32)] * (2 * NBUF)
            + [pltpu.SemaphoreType.DMA] * (2 * NBUF)
        ),
    )
    return f(user_ids.astype(jnp.int32), item_ids.astype(jnp.int32),
             user_table.T, item_table.T)


# speedup vs baseline: 4.0382x; 1.0225x over previous
import jax
import jax.numpy as jnp
from jax import lax
from jax.experimental import pallas as pl
from jax.experimental.pallas import tpu as pltpu
from jax.experimental.pallas import tpu_sc as plsc

BATCH = 16384
EMBED_DIM = 32
SLAB = 128
NUM_CORES = 2
NUM_SUBCORES = 16
NUM_WORKERS = NUM_CORES * NUM_SUBCORES
B_PER_W = BATCH // NUM_WORKERS
LANES = 16
GROUPS = B_PER_W // LANES
NBUF = 8


def _body(user_ids_hbm, item_ids_hbm, user_t_hbm, item_t_hbm, out_hbm,
          uid_v, iid_v, stage_v, out_v,
          ubuf0, ubuf1, ubuf2, ubuf3, ubuf4, ubuf5, ubuf6, ubuf7,
          vbuf0, vbuf1, vbuf2, vbuf3, vbuf4, vbuf5, vbuf6, vbuf7,
          usem0, usem1, usem2, usem3, usem4, usem5, usem6, usem7,
          vsem0, vsem1, vsem2, vsem3, vsem4, vsem5, vsem6, vsem7):
    wid = lax.axis_index("s") * NUM_CORES + lax.axis_index("c")
    base = wid * B_PER_W

    pltpu.sync_copy(user_ids_hbm.at[pl.ds(base, B_PER_W)], uid_v)
    pltpu.sync_copy(item_ids_hbm.at[pl.ds(base, B_PER_W)], iid_v)

    ubufs = [ubuf0, ubuf1, ubuf2, ubuf3, ubuf4, ubuf5, ubuf6, ubuf7]
    vbufs = [vbuf0, vbuf1, vbuf2, vbuf3, vbuf4, vbuf5, vbuf6, vbuf7]
    usems = [usem0, usem1, usem2, usem3, usem4, usem5, usem6, usem7]
    vsems = [vsem0, vsem1, vsem2, vsem3, vsem4, vsem5, vsem6, vsem7]

    lane = lax.iota(jnp.int32, LANES)

    def fire(slot, r_u, r_v):
        ju = pl.multiple_of((r_u >> 7) * SLAB, SLAB)
        jv = pl.multiple_of((r_v >> 7) * SLAB, SLAB)
        pltpu.make_async_copy(
            user_t_hbm.at[:, pl.ds(ju, SLAB)], ubufs[slot],
            usems[slot]).start()
        pltpu.make_async_copy(
            item_t_hbm.at[:, pl.ds(jv, SLAB)], vbufs[slot],
            vsems[slot]).start()

    def waitbuf(slot):
        pltpu.make_async_copy(
            user_t_hbm.at[:, pl.ds(0, SLAB)], ubufs[slot],
            usems[slot]).wait()
        pltpu.make_async_copy(
            item_t_hbm.at[:, pl.ds(0, SLAB)], vbufs[slot],
            vsems[slot]).wait()

    u16p = uid_v[pl.ds(0, LANES)]
    v16p = iid_v[pl.ds(0, LANES)]
    for j in range(NBUF):
        fire(j, u16p[j], v16p[j])

    def group(g, carry):
        g0 = g * LANES
        u16 = uid_v[pl.ds(g0, LANES)]
        v16 = iid_v[pl.ds(g0, LANES)]
        gn0 = jnp.minimum(g + 1, GROUPS - 1) * LANES
        u16n = uid_v[pl.ds(gn0, LANES)]
        v16n = iid_v[pl.ds(gn0, LANES)]
        for j in range(LANES):
            slot = j % NBUF
            jf = j + NBUF
            waitbuf(slot)
            cu = u16[j] & (SLAB - 1)
            cv = v16[j] & (SLAB - 1)
            cu_v = jnp.full((LANES,), cu, jnp.int32)
            cv_v = jnp.full((LANES,), cv, jnp.int32)
            ua = plsc.load_gather(ubufs[slot], [lane, cu_v])
            ub = plsc.load_gather(ubufs[slot], [lane + LANES, cu_v])
            va = plsc.load_gather(vbufs[slot], [lane, cv_v])
            vb = plsc.load_gather(vbufs[slot], [lane + LANES, cv_v])

            if jf < LANES:
                fire(slot, u16[jf], v16[jf])
            else:
                @pl.when(g + 1 < GROUPS)
                def _():
                    fire(slot, u16n[jf - LANES], v16n[jf - LANES])

            stage_v[pl.ds(j * LANES, LANES)] = ua * va + ub * vb

        acc = jnp.zeros((LANES,), jnp.float32)
        for l in range(LANES):
            acc = acc + plsc.load_gather(
                stage_v, [lane * LANES + l])
        out_v[pl.ds(g0, LANES)] = acc
        return carry

    lax.fori_loop(0, GROUPS, group, 0)

    pltpu.sync_copy(out_v, out_hbm.at[pl.ds(base, B_PER_W)])


@jax.jit
def kernel(user_ids, item_ids, user_table, item_table):
    mesh = plsc.VectorSubcoreMesh(core_axis_name="c", subcore_axis_name="s")
    f = pl.kernel(
        _body,
        mesh=mesh,
        compiler_params=pltpu.CompilerParams(needs_layout_passes=False),
        out_type=jax.ShapeDtypeStruct((BATCH,), jnp.float32),
        scratch_types=(
            [pltpu.VMEM((B_PER_W,), jnp.int32),
             pltpu.VMEM((B_PER_W,), jnp.int32),
             pltpu.VMEM((LANES * LANES,), jnp.float32),
             pltpu.VMEM((B_PER_W,), jnp.float32)]
            + [pltpu.VMEM((EMBED_DIM, SLAB), jnp.float32)] * (2 * NBUF)
            + [pltpu.SemaphoreType.DMA] * (2 * NBUF)
        ),
    )
    return f(user_ids.astype(jnp.int32), item_ids.astype(jnp.int32),
             user_table.T, item_table.T)
